# Initial kernel scaffold; baseline (speedup 1.0000x reference)
import jax
import jax.numpy as jnp
from jax.experimental import pallas as pl


def kernel(z_new, t_new, e_new, b_new, z_buf, t_buf, e_buf, b_buf):
    raise NotImplementedError("write your pallas kernel here")


# speedup vs baseline: 4.3185x; 4.3185x over previous
import functools

import jax
import jax.numpy as jnp
from jax import lax
from jax.experimental import pallas as pl
from jax.experimental.pallas import tpu as pltpu
from jax.experimental.pallas import tpu_sc as plsc

_K = 65536
_DIM = 128
_B = 16384
_PTR = 60000
_SIZE = 0
_TAIL = _K - _PTR
_WRAP = _B - _TAIL
_MID = _PTR - _WRAP

_NC = 2
_NS = 16
_NW = _NC * _NS
_CHUNK = 32768


def _build_tasks():
    segs = [
        ("z_new", _TAIL * _DIM, "z", 0, _WRAP * _DIM),
        ("z_buf", _WRAP * _DIM, "z", _WRAP * _DIM, _MID * _DIM),
        ("z_new", 0, "z", _PTR * _DIM, _TAIL * _DIM),
    ]
    for name in ("t", "e", "b"):
        segs += [
            (name + "_new", _TAIL, name, 0, _WRAP),
            (name + "_buf", _WRAP, name, _WRAP, _MID),
            (name + "_new", 0, name, _PTR, _TAIL),
        ]
    total = sum(s[4] for s in segs)
    assert total % _NW == 0 and (total // _NW) % 8 == 0
    quota = total // _NW
    tasks = [[] for _ in range(_NW)]
    w, room = 0, quota
    for src, so, dst, do, n in segs:
        while n > 0:
            take = min(n, room, _CHUNK)
            tasks[w].append((src, so, dst, do, take))
            so += take
            do += take
            n -= take
            room -= take
            if room == 0 and w + 1 < _NW:
                w, room = w + 1, quota
    return tasks

_TASKS = _build_tasks()


def _enqueue_body(z_new, t_new, e_new, b_new, z_buf, t_buf, e_buf, b_buf,
                  z_out, t_out, e_out, b_out,
                  vm0, vm1, in0, in1, out0, out1):
    wid = lax.axis_index("s") * _NC + lax.axis_index("c")
    refs = {
        "z_new": z_new, "t_new": t_new, "e_new": e_new, "b_new": b_new,
        "z_buf": z_buf, "t_buf": t_buf, "e_buf": e_buf, "b_buf": b_buf,
        "z": z_out, "t": t_out, "e": e_out, "b": b_out,
    }
    bufs, in_sems, out_sems = (vm0, vm1), (in0, in1), (out0, out1)
    for w, chunks in enumerate(_TASKS):
        @pl.when(wid == w)
        def _(chunks=chunks):
            cps = []
            for i, (src, so, dst, do, n) in enumerate(chunks):
                buf = bufs[i % 2].at[pl.ds(0, n)]
                cps.append((
                    pltpu.make_async_copy(
                        refs[src].at[pl.ds(so, n)], buf, in_sems[i % 2]),
                    pltpu.make_async_copy(
                        buf, refs[dst].at[pl.ds(do, n)], out_sems[i % 2]),
                ))
            nch = len(cps)
            cps[0][0].start()
            for i in range(nch):
                cin, cout = cps[i]
                cin.wait()
                cout.start()
                if i + 1 < nch:
                    if i >= 1:
                        cps[i - 1][1].wait()
                    cps[i + 1][0].start()
            for j in range(max(0, nch - 2), nch):
                cps[j][1].wait()


@functools.cache
def _make_enqueue():
    return pl.kernel(
        _enqueue_body,
        out_type=(
            jax.ShapeDtypeStruct((_K * _DIM,), jnp.float32),
            jax.ShapeDtypeStruct((_K,), jnp.float32),
            jax.ShapeDtypeStruct((_K,), jnp.float32),
            jax.ShapeDtypeStruct((_K,), jnp.float32),
        ),
        mesh=plsc.VectorSubcoreMesh(
            core_axis_name="c", subcore_axis_name="s",
            num_cores=_NC, num_subcores=_NS),
        scratch_types=[
            pltpu.VMEM((_CHUNK,), jnp.float32),
            pltpu.VMEM((_CHUNK,), jnp.float32),
            pltpu.SemaphoreType.DMA,
            pltpu.SemaphoreType.DMA,
            pltpu.SemaphoreType.DMA,
            pltpu.SemaphoreType.DMA,
        ],
    )


def kernel(z_new, t_new, e_new, b_new, z_buf, t_buf, e_buf, b_buf):
    z, t, e, b = _make_enqueue()(
        z_new.reshape(_B * _DIM), t_new, e_new,
        lax.bitcast_convert_type(b_new, jnp.float32),
        z_buf.reshape(_K * _DIM), t_buf, e_buf,
        lax.bitcast_convert_type(b_buf, jnp.float32))
    new_ptr = jnp.asarray((_PTR + _B) % _K, dtype=jnp.int32)
    new_size = jnp.asarray(min(_SIZE + _B, _K), dtype=jnp.int32)
    return (z.reshape(_K, _DIM), t, e,
            lax.bitcast_convert_type(b, jnp.int32), new_ptr, new_size)


# speedup vs baseline: 4.4509x; 1.0307x over previous
import functools

import jax
import jax.numpy as jnp
from jax import lax
from jax.experimental import pallas as pl
from jax.experimental.pallas import tpu as pltpu
from jax.experimental.pallas import tpu_sc as plsc

_K = 65536
_DIM = 128
_B = 16384
_PTR = 60000
_SIZE = 0
_TAIL = _K - _PTR
_WRAP = _B - _TAIL
_MID = _PTR - _WRAP

_NC = 2
_NS = 16
_NW = _NC * _NS
_CHUNK = 63488


def _build_tasks():
    segs = [
        ("z_new", _TAIL * _DIM, "z", 0, _WRAP * _DIM),
        ("z_buf", _WRAP * _DIM, "z", _WRAP * _DIM, _MID * _DIM),
        ("z_new", 0, "z", _PTR * _DIM, _TAIL * _DIM),
    ]
    for name in ("t", "e", "b"):
        segs += [
            (name + "_new", _TAIL, name, 0, _WRAP),
            (name + "_buf", _WRAP, name, _WRAP, _MID),
            (name + "_new", 0, name, _PTR, _TAIL),
        ]
    total = sum(s[4] for s in segs)
    assert total % _NW == 0 and (total // _NW) % 8 == 0
    quota = total // _NW
    tasks = [[] for _ in range(_NW)]
    w, room = 0, quota
    for src, so, dst, do, n in segs:
        while n > 0:
            take = min(n, room, _CHUNK)
            tasks[w].append((src, so, dst, do, take))
            so += take
            do += take
            n -= take
            room -= take
            if room == 0 and w + 1 < _NW:
                w, room = w + 1, quota
    return tasks

_TASKS = _build_tasks()


def _enqueue_body(z_new, t_new, e_new, b_new, z_buf, t_buf, e_buf, b_buf,
                  z_out, t_out, e_out, b_out,
                  vm0, vm1, in0, in1, out0, out1):
    wid = lax.axis_index("s") * _NC + lax.axis_index("c")
    refs = {
        "z_new": z_new, "t_new": t_new, "e_new": e_new, "b_new": b_new,
        "z_buf": z_buf, "t_buf": t_buf, "e_buf": e_buf, "b_buf": b_buf,
        "z": z_out, "t": t_out, "e": e_out, "b": b_out,
    }
    bufs, in_sems, out_sems = (vm0, vm1), (in0, in1), (out0, out1)
    for w, chunks in enumerate(_TASKS):
        @pl.when(wid == w)
        def _(chunks=chunks):
            cps = []
            for i, (src, so, dst, do, n) in enumerate(chunks):
                buf = bufs[i % 2].at[pl.ds(0, n)]
                cps.append((
                    pltpu.make_async_copy(
                        refs[src].at[pl.ds(so, n)], buf, in_sems[i % 2]),
                    pltpu.make_async_copy(
                        buf, refs[dst].at[pl.ds(do, n)], out_sems[i % 2]),
                ))
            nch = len(cps)
            cps[0][0].start()
            for i in range(nch):
                cin, cout = cps[i]
                cin.wait()
                cout.start()
                if i + 1 < nch:
                    if i >= 1:
                        cps[i - 1][1].wait()
                    cps[i + 1][0].start()
            for j in range(max(0, nch - 2), nch):
                cps[j][1].wait()


@functools.cache
def _make_enqueue():
    return pl.kernel(
        _enqueue_body,
        out_type=(
            jax.ShapeDtypeStruct((_K * _DIM,), jnp.float32),
            jax.ShapeDtypeStruct((_K,), jnp.float32),
            jax.ShapeDtypeStruct((_K,), jnp.float32),
            jax.ShapeDtypeStruct((_K,), jnp.float32),
        ),
        mesh=plsc.VectorSubcoreMesh(
            core_axis_name="c", subcore_axis_name="s",
            num_cores=_NC, num_subcores=_NS),
        scratch_types=[
            pltpu.VMEM((_CHUNK,), jnp.float32),
            pltpu.VMEM((_CHUNK,), jnp.float32),
            pltpu.SemaphoreType.DMA,
            pltpu.SemaphoreType.DMA,
            pltpu.SemaphoreType.DMA,
            pltpu.SemaphoreType.DMA,
        ],
    )


def kernel(z_new, t_new, e_new, b_new, z_buf, t_buf, e_buf, b_buf):
    z, t, e, b = _make_enqueue()(
        z_new.reshape(_B * _DIM), t_new, e_new,
        lax.bitcast_convert_type(b_new, jnp.float32),
        z_buf.reshape(_K * _DIM), t_buf, e_buf,
        lax.bitcast_convert_type(b_buf, jnp.float32))
    new_ptr = jnp.asarray((_PTR + _B) % _K, dtype=jnp.int32)
    new_size = jnp.asarray(min(_SIZE + _B, _K), dtype=jnp.int32)
    return (z.reshape(_K, _DIM), t, e,
            lax.bitcast_convert_type(b, jnp.int32), new_ptr, new_size)


# speedup vs baseline: 4.8837x; 1.0972x over previous
import functools

import jax
import jax.numpy as jnp
from jax import lax
from jax.experimental import pallas as pl
from jax.experimental.pallas import tpu as pltpu
from jax.experimental.pallas import tpu_sc as plsc

_K = 65536
_DIM = 128
_B = 16384
_PTR = 60000
_SIZE = 0
_TAIL = _K - _PTR
_WRAP = _B - _TAIL
_MID = _PTR - _WRAP

_NC = 2
_NS = 16
_NW = _NC * _NS
_CHUNK = 63488


def _build_tasks():
    z_segs = [
        ("z_new", _TAIL * _DIM, "z", 0, _WRAP * _DIM),
        ("z_buf", _WRAP * _DIM, "z", _WRAP * _DIM, _MID * _DIM),
        ("z_new", 0, "z", _PTR * _DIM, _TAIL * _DIM),
    ]
    small_segs = []
    for name in ("t", "e", "b"):
        small_segs += [
            (name + "_new", _TAIL, name, 0, _WRAP),
            (name + "_buf", _WRAP, name, _WRAP, _MID),
            (name + "_new", 0, name, _PTR, _TAIL),
        ]
    tasks = [[] for _ in range(_NW)]
    z_total = sum(s[4] for s in z_segs)
    assert z_total % _NW == 0 and (z_total // _NW) % 8 == 0
    quota = z_total // _NW
    w, room = 0, quota
    for src, so, dst, do, n in z_segs:
        while n > 0:
            take = min(n, room, _CHUNK)
            tasks[w].append((src, so, dst, do, take))
            so += take
            do += take
            n -= take
            room -= take
            if room == 0 and w + 1 < _NW:
                w, room = w + 1, quota
    pieces = []
    for src, so, dst, do, n in small_segs:
        while n > 0:
            take = min(n, 8192)
            pieces.append((src, so, dst, do, take))
            so += take
            do += take
            n -= take
    assert len(pieces) <= _NW
    for i, p in enumerate(pieces):
        tasks[i].append(p)
    return tasks

_TASKS = _build_tasks()


def _enqueue_body(z_new, t_new, e_new, b_new, z_buf, t_buf, e_buf, b_buf,
                  z_out, t_out, e_out, b_out,
                  vm0, vm1, in0, in1, out0, out1):
    wid = lax.axis_index("s") * _NC + lax.axis_index("c")
    refs = {
        "z_new": z_new, "t_new": t_new, "e_new": e_new, "b_new": b_new,
        "z_buf": z_buf, "t_buf": t_buf, "e_buf": e_buf, "b_buf": b_buf,
        "z": z_out, "t": t_out, "e": e_out, "b": b_out,
    }
    bufs, in_sems, out_sems = (vm0, vm1), (in0, in1), (out0, out1)
    for w, chunks in enumerate(_TASKS):
        @pl.when(wid == w)
        def _(chunks=chunks):
            cps = []
            for i, (src, so, dst, do, n) in enumerate(chunks):
                buf = bufs[i % 2].at[pl.ds(0, n)]
                cps.append((
                    pltpu.make_async_copy(
                        refs[src].at[pl.ds(so, n)], buf, in_sems[i % 2]),
                    pltpu.make_async_copy(
                        buf, refs[dst].at[pl.ds(do, n)], out_sems[i % 2]),
                ))
            nch = len(cps)
            cps[0][0].start()
            for i in range(nch):
                cin, cout = cps[i]
                cin.wait()
                cout.start()
                if i + 1 < nch:
                    if i >= 1:
                        cps[i - 1][1].wait()
                    cps[i + 1][0].start()
            for j in range(max(0, nch - 2), nch):
                cps[j][1].wait()


@functools.cache
def _make_enqueue():
    return pl.kernel(
        _enqueue_body,
        out_type=(
            jax.ShapeDtypeStruct((_K * _DIM,), jnp.float32),
            jax.ShapeDtypeStruct((_K,), jnp.float32),
            jax.ShapeDtypeStruct((_K,), jnp.float32),
            jax.ShapeDtypeStruct((_K,), jnp.float32),
        ),
        mesh=plsc.VectorSubcoreMesh(
            core_axis_name="c", subcore_axis_name="s",
            num_cores=_NC, num_subcores=_NS),
        scratch_types=[
            pltpu.VMEM((_CHUNK,), jnp.float32),
            pltpu.VMEM((_CHUNK,), jnp.float32),
            pltpu.SemaphoreType.DMA,
            pltpu.SemaphoreType.DMA,
            pltpu.SemaphoreType.DMA,
            pltpu.SemaphoreType.DMA,
        ],
    )


def kernel(z_new, t_new, e_new, b_new, z_buf, t_buf, e_buf, b_buf):
    z, t, e, b = _make_enqueue()(
        z_new.reshape(_B * _DIM), t_new, e_new,
        lax.bitcast_convert_type(b_new, jnp.float32),
        z_buf.reshape(_K * _DIM), t_buf, e_buf,
        lax.bitcast_convert_type(b_buf, jnp.float32))
    new_ptr = jnp.asarray((_PTR + _B) % _K, dtype=jnp.int32)
    new_size = jnp.asarray(min(_SIZE + _B, _K), dtype=jnp.int32)
    return (z.reshape(_K, _DIM), t, e,
            lax.bitcast_convert_type(b, jnp.int32), new_ptr, new_size)
